# baseline (speedup 1.0000x reference)
import numpy as np

import jax
import jax.numpy as jnp
from jax.experimental import pallas as pl
from jax.experimental.pallas import tpu as pltpu

B = 32
G = 16
NG = B // G

_L1 = (5, 3, 16, 32, 36)
_L2 = (5, 16, 32, 16, 20)
_L3 = (3, 32, 64, 8, 10)


def _build_T(w, K, Cin, Cout, Wout, Wpos):
    w4 = w.reshape(K, K, Cin, Cout).astype(jnp.float32)
    j = np.arange(K)[:, None, None]
    wo = np.arange(Wout)[None, :, None]
    wp = np.arange(Wpos)[None, None, :]
    O = (wp == wo + j).astype(np.float32)
    T = jnp.einsum('jwp,ijcd->ipcwd', O, w4)
    return T.reshape(K, Wpos * Cin, Wout * Cout).astype(jnp.bfloat16)


def _rowsel(H, parity):
    S = np.zeros((G * H // 2, G * H), np.float32)
    for b in range(G):
        for q in range(H // 2):
            S[b * (H // 2) + q, b * H + 2 * q + parity] = 1.0
    return S


def _laneshift(W, C):
    R = np.zeros((W * C, W * C), np.float32)
    for w in range(W - 1):
        for c in range(C):
            R[(w + 1) * C + c, w * C + c] = 1.0
    return R


def _build_sel(Wout_pooled, C, halo, in_lanes, out_lanes):
    S = np.zeros((in_lanes, out_lanes), np.float32)
    for w in range(Wout_pooled):
        for c in range(C):
            S[(2 * w) * C + c, (halo + w) * C + c] = 1.0
    return S


def _tapshift(H, K, halo):
    S = np.zeros((K, G * H, G * H), np.float32)
    for i in range(K):
        for b in range(G):
            for q in range(H):
                r = q + i - halo
                if 0 <= r < H:
                    S[i, b * H + q, b * H + r] = 1.0
    return S


_SC1 = np.concatenate([_rowsel(32, 0), _rowsel(32, 1)], axis=0)
_SC2 = np.concatenate([_rowsel(16, 0), _rowsel(16, 1)], axis=0)
_SC3 = np.concatenate([_rowsel(8, 0), _rowsel(8, 1)], axis=0)
_R3 = _laneshift(8, 64)
_SEL1 = _build_sel(16, 16, 2, 512, 320)
_SEL2 = _build_sel(8, 32, 1, 512, 320)
_RSEL1 = _laneshift(32, 16) @ _SEL1
_RSEL2 = _laneshift(16, 32) @ _SEL2
_SH2 = _tapshift(16, 5, 2)


def _vpool(y, sc_ref):
    yb = y.astype(jnp.bfloat16)
    ms = jnp.dot(sc_ref[...], yb, preferred_element_type=jnp.float32)
    half = ms.shape[0] // 2
    return jnp.maximum(ms[:half], ms[half:]).astype(jnp.bfloat16)


def _hpool_sel(mv, sela_ref, selb_ref):
    f32 = jnp.float32
    return jnp.maximum(
        jnp.dot(mv, sela_ref[...], preferred_element_type=f32),
        jnp.dot(mv, selb_ref[...], preferred_element_type=f32))


def _cnn_body(xt_ref, t1_ref, sc1_ref, sh1_ref,
              sc1m_ref, sel1_ref, rsel1_ref, sh2m_ref,
              t2_ref, sc2_ref, sh2_ref,
              sc2m_ref, sel2_ref, rsel2_ref,
              t3_ref, sc3_ref, sh3_ref,
              sc3m_ref, r3_ref,
              wf1_ref, fb1_ref, fw2_ref, fb2_ref,
              o_ref, a25_ref, a3_ref):
    f32 = jnp.float32

    acc = None
    for i in range(5):
        lhs = xt_ref[:, i:i + 32, :].reshape(B * 32, 108)
        d = jnp.dot(lhs, t1_ref[i], preferred_element_type=f32)
        acc = d if acc is None else acc + d
    y = jnp.maximum(acc * sc1_ref[...] + sh1_ref[...], 0.0)
    for g in range(NG):
        mv = _vpool(y[g * G * 32:(g + 1) * G * 32], sc1m_ref)
        p = _hpool_sel(mv, sel1_ref, rsel1_ref)
        pb = p.astype(jnp.bfloat16)
        for i in range(5):
            pi = jnp.dot(sh2m_ref[i], pb, preferred_element_type=f32)
            a25_ref[i, g * G:(g + 1) * G] = (
                pi.astype(jnp.bfloat16).reshape(G, 16, 320))

    acc = None
    for i in range(5):
        lhs = a25_ref[i].reshape(B * 16, 320)
        d = jnp.dot(lhs, t2_ref[i], preferred_element_type=f32)
        acc = d if acc is None else acc + d
    y = jnp.maximum(acc * sc2_ref[...] + sh2_ref[...], 0.0)
    a3_ref[:, 0:1, :] = jnp.zeros((B, 1, 320), f32)
    a3_ref[:, 9:10, :] = jnp.zeros((B, 1, 320), f32)
    for g in range(NG):
        mv = _vpool(y[g * G * 16:(g + 1) * G * 16], sc2m_ref)
        p = _hpool_sel(mv, sel2_ref, rsel2_ref)
        a3_ref[g * G:(g + 1) * G, 1:9, :] = p.reshape(G, 8, 320)

    acc = None
    for i in range(3):
        lhs = a3_ref[:, i:i + 8, :].reshape(B * 8, 320).astype(jnp.bfloat16)
        d = jnp.dot(lhs, t3_ref[i], preferred_element_type=f32)
        acc = d if acc is None else acc + d
    y = jnp.maximum(acc * sc3_ref[...] + sh3_ref[...], 0.0)
    hs = []
    for g in range(NG):
        mv = _vpool(y[g * G * 8:(g + 1) * G * 8], sc3m_ref)
        mr = jnp.dot(mv, r3_ref[...], preferred_element_type=f32)
        hs.append(jnp.maximum(mv.astype(f32), mr).astype(jnp.bfloat16))
    h = hs[0] if NG == 1 else jnp.concatenate(hs, axis=0)

    h = h.reshape(B, 4, 512)
    f = None
    for hh in range(4):
        d = jnp.dot(h[:, hh, :], wf1_ref[hh],
                    preferred_element_type=f32)
        f = d if f is None else f + d
    hrelu = jnp.maximum(f + fb1_ref[...], 0.0)
    o_ref[...] = jnp.dot(hrelu, fw2_ref[...],
                         preferred_element_type=f32) + fb2_ref[...]


def kernel(x, w1, s1, t1, w2, s2, t2, w3, s3, t3, fw1, fb1, fw2, fb2):
    n = x.shape[0]

    xt = jnp.transpose(x, (0, 2, 3, 1))
    xt = jnp.pad(xt, ((0, 0), (2, 2), (2, 2), (0, 0)))
    xt = xt.reshape(n, 36, 108).astype(jnp.bfloat16)

    T1 = _build_T(w1, *_L1)
    T2 = _build_T(w2, *_L2)
    T3 = _build_T(w3, *_L3)
    sc1, sh1 = jnp.tile(s1, (1, 32)), jnp.tile(t1, (1, 32))
    sc2, sh2 = jnp.tile(s2, (1, 16)), jnp.tile(t2, (1, 16))
    sc3, sh3 = jnp.tile(s3, (1, 8)), jnp.tile(t3, (1, 8))

    f4 = fw1.reshape(4, 4, 64, 256)
    Wf1 = jnp.zeros((4, 8, 64, 256), fw1.dtype).at[:, 0::2].set(f4)
    Wf1 = Wf1.reshape(4, 512, 256)

    bf = jnp.bfloat16
    sc1m, sc2m, sc3m = (jnp.asarray(_SC1, bf), jnp.asarray(_SC2, bf),
                        jnp.asarray(_SC3, bf))
    sel1, sel2 = jnp.asarray(_SEL1, bf), jnp.asarray(_SEL2, bf)
    rsel1, rsel2 = jnp.asarray(_RSEL1, bf), jnp.asarray(_RSEL2, bf)
    r3 = jnp.asarray(_R3, bf)
    sh2m = jnp.asarray(_SH2, bf)

    full = lambda shape: pl.BlockSpec(shape, lambda i: tuple(0 for _ in shape))
    in_specs = [
        pl.BlockSpec((B, 36, 108), lambda i: (i, 0, 0)),
        full((5, 108, 512)), full((1, 512)), full((1, 512)),
        full((512, 512)), full((512, 320)), full((512, 320)), full((5, 256, 256)),
        full((5, 320, 512)), full((1, 512)), full((1, 512)),
        full((256, 256)), full((512, 320)), full((512, 320)),
        full((3, 320, 512)), full((1, 512)), full((1, 512)),
        full((128, 128)), full((512, 512)),
        full((4, 512, 256)), full((1, 256)), full((256, 2)), full((1, 2)),
    ]
    out = pl.pallas_call(
        _cnn_body,
        out_shape=jax.ShapeDtypeStruct((n, 2), jnp.float32),
        grid=(n // B,),
        in_specs=in_specs,
        out_specs=pl.BlockSpec((B, 2), lambda i: (i, 0)),
        scratch_shapes=[
            pltpu.VMEM((5, B, 16, 320), jnp.bfloat16),
            pltpu.VMEM((B, 10, 320), jnp.float32),
        ],
        compiler_params=pltpu.CompilerParams(
            dimension_semantics=("arbitrary",)),
    )(xt, T1, sc1, sh1, sc1m, sel1, rsel1, sh2m,
      T2, sc2, sh2, sc2m, sel2, rsel2,
      T3, sc3, sh3, sc3m, r3,
      Wf1, fb1, fw2, fb2)
    return out


# speedup vs baseline: 1.1838x; 1.1838x over previous
import numpy as np

import jax
import jax.numpy as jnp
from jax.experimental import pallas as pl
from jax.experimental.pallas import tpu as pltpu

B = 64
G = 16
NG = B // G

_L1 = (5, 3, 16, 32, 36)
_L2 = (5, 16, 32, 16, 20)
_L3 = (3, 32, 64, 8, 10)


def _build_T(w, K, Cin, Cout, Wout, Wpos):
    w4 = w.reshape(K, K, Cin, Cout).astype(jnp.float32)
    j = np.arange(K)[:, None, None]
    wo = np.arange(Wout)[None, :, None]
    wp = np.arange(Wpos)[None, None, :]
    O = (wp == wo + j).astype(np.float32)
    T = jnp.einsum('jwp,ijcd->ipcwd', O, w4)
    return T.reshape(K, Wpos * Cin, Wout * Cout).astype(jnp.bfloat16)


def _rowsel(H, parity):
    S = np.zeros((G * H // 2, G * H), np.float32)
    for b in range(G):
        for q in range(H // 2):
            S[b * (H // 2) + q, b * H + 2 * q + parity] = 1.0
    return S


def _laneshift(W, C):
    R = np.zeros((W * C, W * C), np.float32)
    for w in range(W - 1):
        for c in range(C):
            R[(w + 1) * C + c, w * C + c] = 1.0
    return R


def _build_sel(Wout_pooled, C, halo, in_lanes, out_lanes):
    S = np.zeros((in_lanes, out_lanes), np.float32)
    for w in range(Wout_pooled):
        for c in range(C):
            S[(2 * w) * C + c, (halo + w) * C + c] = 1.0
    return S


_SC1 = np.concatenate([_rowsel(32, 0), _rowsel(32, 1)], axis=0)
_SC2 = np.concatenate([_rowsel(16, 0), _rowsel(16, 1)], axis=0)
_SC3 = np.concatenate([_rowsel(8, 0), _rowsel(8, 1)], axis=0)
_R3 = _laneshift(8, 64)
_SEL1 = _build_sel(16, 16, 2, 512, 320)
_SEL2 = _build_sel(8, 32, 1, 512, 320)
_RSEL1 = _laneshift(32, 16) @ _SEL1
_RSEL2 = _laneshift(16, 32) @ _SEL2


def _vpool(y, sc_ref):
    yb = y.astype(jnp.bfloat16)
    ms = jnp.dot(sc_ref[...], yb, preferred_element_type=jnp.float32)
    half = ms.shape[0] // 2
    return jnp.maximum(ms[:half], ms[half:]).astype(jnp.bfloat16)


def _hpool_sel(mv, sela_ref, selb_ref):
    f32 = jnp.float32
    return jnp.maximum(
        jnp.dot(mv, sela_ref[...], preferred_element_type=f32),
        jnp.dot(mv, selb_ref[...], preferred_element_type=f32))


def _cnn_body(xt_ref, t1_ref, sc1_ref, sh1_ref,
              sc1m_ref, sel1_ref, rsel1_ref,
              t2_ref, sc2_ref, sh2_ref,
              sc2m_ref, sel2_ref, rsel2_ref,
              t3_ref, sc3_ref, sh3_ref,
              sc3m_ref, r3_ref,
              wf1_ref, fb1_ref, fw2_ref, fb2_ref,
              o_ref, a2_ref, a3_ref):
    f32 = jnp.float32

    acc = None
    for i in range(5):
        lhs = xt_ref[:, i:i + 32, :].reshape(B * 32, 108)
        d = jnp.dot(lhs, t1_ref[i], preferred_element_type=f32)
        acc = d if acc is None else acc + d
    y = jnp.maximum(acc * sc1_ref[...] + sh1_ref[...], 0.0)
    a2_ref[:, 0:2, :] = jnp.zeros((B, 2, 320), jnp.bfloat16)
    a2_ref[:, 18:20, :] = jnp.zeros((B, 2, 320), jnp.bfloat16)
    for g in range(NG):
        mv = _vpool(y[g * G * 32:(g + 1) * G * 32], sc1m_ref)
        p = _hpool_sel(mv, sel1_ref, rsel1_ref)
        a2_ref[g * G:(g + 1) * G, 2:18, :] = (
            p.astype(jnp.bfloat16).reshape(G, 16, 320))

    acc = None
    for i in range(5):
        lhs = a2_ref[:, i:i + 16, :].reshape(B * 16, 320)
        d = jnp.dot(lhs, t2_ref[i], preferred_element_type=f32)
        acc = d if acc is None else acc + d
    y = jnp.maximum(acc * sc2_ref[...] + sh2_ref[...], 0.0)
    a3_ref[:, 0:1, :] = jnp.zeros((B, 1, 320), f32)
    a3_ref[:, 9:10, :] = jnp.zeros((B, 1, 320), f32)
    for g in range(NG):
        mv = _vpool(y[g * G * 16:(g + 1) * G * 16], sc2m_ref)
        p = _hpool_sel(mv, sel2_ref, rsel2_ref)
        a3_ref[g * G:(g + 1) * G, 1:9, :] = p.reshape(G, 8, 320)

    acc = None
    for i in range(3):
        lhs = a3_ref[:, i:i + 8, :].reshape(B * 8, 320).astype(jnp.bfloat16)
        d = jnp.dot(lhs, t3_ref[i], preferred_element_type=f32)
        acc = d if acc is None else acc + d
    y = jnp.maximum(acc * sc3_ref[...] + sh3_ref[...], 0.0)
    hs = []
    for g in range(NG):
        mv = _vpool(y[g * G * 8:(g + 1) * G * 8], sc3m_ref)
        mr = jnp.dot(mv, r3_ref[...], preferred_element_type=f32)
        hs.append(jnp.maximum(mv.astype(f32), mr).astype(jnp.bfloat16))
    h = hs[0] if NG == 1 else jnp.concatenate(hs, axis=0)

    h = h.reshape(B, 4, 512)
    f = None
    for hh in range(4):
        d = jnp.dot(h[:, hh, :], wf1_ref[hh],
                    preferred_element_type=f32)
        f = d if f is None else f + d
    hrelu = jnp.maximum(f + fb1_ref[...], 0.0)
    o_ref[...] = jnp.dot(hrelu, fw2_ref[...],
                         preferred_element_type=f32) + fb2_ref[...]


def kernel(x, w1, s1, t1, w2, s2, t2, w3, s3, t3, fw1, fb1, fw2, fb2):
    n = x.shape[0]

    xt = jnp.transpose(x, (0, 2, 3, 1))
    xt = jnp.pad(xt, ((0, 0), (2, 2), (2, 2), (0, 0)))
    xt = xt.reshape(n, 36, 108).astype(jnp.bfloat16)

    T1 = _build_T(w1, *_L1)
    T2 = _build_T(w2, *_L2)
    T3 = _build_T(w3, *_L3)
    sc1, sh1 = jnp.tile(s1, (1, 32)), jnp.tile(t1, (1, 32))
    sc2, sh2 = jnp.tile(s2, (1, 16)), jnp.tile(t2, (1, 16))
    sc3, sh3 = jnp.tile(s3, (1, 8)), jnp.tile(t3, (1, 8))

    f4 = fw1.reshape(4, 4, 64, 256)
    Wf1 = jnp.zeros((4, 8, 64, 256), fw1.dtype).at[:, 0::2].set(f4)
    Wf1 = Wf1.reshape(4, 512, 256)

    bf = jnp.bfloat16
    sc1m, sc2m, sc3m = (jnp.asarray(_SC1, bf), jnp.asarray(_SC2, bf),
                        jnp.asarray(_SC3, bf))
    sel1, sel2 = jnp.asarray(_SEL1, bf), jnp.asarray(_SEL2, bf)
    rsel1, rsel2 = jnp.asarray(_RSEL1, bf), jnp.asarray(_RSEL2, bf)
    r3 = jnp.asarray(_R3, bf)

    full = lambda shape: pl.BlockSpec(shape, lambda i: tuple(0 for _ in shape))
    in_specs = [
        pl.BlockSpec((B, 36, 108), lambda i: (i, 0, 0)),
        full((5, 108, 512)), full((1, 512)), full((1, 512)),
        full((512, 512)), full((512, 320)), full((512, 320)),
        full((5, 320, 512)), full((1, 512)), full((1, 512)),
        full((256, 256)), full((512, 320)), full((512, 320)),
        full((3, 320, 512)), full((1, 512)), full((1, 512)),
        full((128, 128)), full((512, 512)),
        full((4, 512, 256)), full((1, 256)), full((256, 2)), full((1, 2)),
    ]
    out = pl.pallas_call(
        _cnn_body,
        out_shape=jax.ShapeDtypeStruct((n, 2), jnp.float32),
        grid=(n // B,),
        in_specs=in_specs,
        out_specs=pl.BlockSpec((B, 2), lambda i: (i, 0)),
        scratch_shapes=[
            pltpu.VMEM((B, 20, 320), jnp.bfloat16),
            pltpu.VMEM((B, 10, 320), jnp.float32),
        ],
        compiler_params=pltpu.CompilerParams(
            dimension_semantics=("arbitrary",)),
    )(xt, T1, sc1, sh1, sc1m, sel1, rsel1,
      T2, sc2, sh2, sc2m, sel2, rsel2,
      T3, sc3, sh3, sc3m, r3,
      Wf1, fb1, fw2, fb2)
    return out


# speedup vs baseline: 1.1908x; 1.0059x over previous
import numpy as np

import jax
import jax.numpy as jnp
from jax.experimental import pallas as pl
from jax.experimental.pallas import tpu as pltpu

B = 128
G = 16
NG = B // G

_L1 = (5, 3, 16, 32, 36)
_L2 = (5, 16, 32, 16, 20)
_L3 = (3, 32, 64, 8, 10)


def _build_T(w, K, Cin, Cout, Wout, Wpos):
    w4 = w.reshape(K, K, Cin, Cout).astype(jnp.float32)
    j = np.arange(K)[:, None, None]
    wo = np.arange(Wout)[None, :, None]
    wp = np.arange(Wpos)[None, None, :]
    O = (wp == wo + j).astype(np.float32)
    T = jnp.einsum('jwp,ijcd->ipcwd', O, w4)
    return T.reshape(K, Wpos * Cin, Wout * Cout).astype(jnp.bfloat16)


def _rowsel(H, parity):
    S = np.zeros((G * H // 2, G * H), np.float32)
    for b in range(G):
        for q in range(H // 2):
            S[b * (H // 2) + q, b * H + 2 * q + parity] = 1.0
    return S


def _laneshift(W, C):
    R = np.zeros((W * C, W * C), np.float32)
    for w in range(W - 1):
        for c in range(C):
            R[(w + 1) * C + c, w * C + c] = 1.0
    return R


def _build_sel(Wout_pooled, C, halo, in_lanes, out_lanes):
    S = np.zeros((in_lanes, out_lanes), np.float32)
    for w in range(Wout_pooled):
        for c in range(C):
            S[(2 * w) * C + c, (halo + w) * C + c] = 1.0
    return S


_SC1 = np.concatenate([_rowsel(32, 0), _rowsel(32, 1)], axis=0)
_SC2 = np.concatenate([_rowsel(16, 0), _rowsel(16, 1)], axis=0)
_SC3 = np.concatenate([_rowsel(8, 0), _rowsel(8, 1)], axis=0)
_R3 = _laneshift(8, 64)
_SEL1 = _build_sel(16, 16, 2, 512, 320)
_SEL2 = _build_sel(8, 32, 1, 512, 320)
_RSEL1 = _laneshift(32, 16) @ _SEL1
_RSEL2 = _laneshift(16, 32) @ _SEL2


def _vpool(y, sc_ref):
    yb = y.astype(jnp.bfloat16)
    ms = jnp.dot(sc_ref[...], yb, preferred_element_type=jnp.float32)
    half = ms.shape[0] // 2
    return jnp.maximum(ms[:half], ms[half:]).astype(jnp.bfloat16)


def _hpool_sel(mv, sela_ref, selb_ref):
    f32 = jnp.float32
    return jnp.maximum(
        jnp.dot(mv, sela_ref[...], preferred_element_type=f32),
        jnp.dot(mv, selb_ref[...], preferred_element_type=f32))


def _cnn_body(xt_ref, t1_ref, sc1_ref, sh1_ref,
              sc1m_ref, sel1_ref, rsel1_ref,
              t2_ref, sc2_ref, sh2_ref,
              sc2m_ref, sel2_ref, rsel2_ref,
              t3_ref, sc3_ref, sh3_ref,
              sc3m_ref, r3_ref,
              wf1_ref, fb1_ref, fw2_ref, fb2_ref,
              o_ref, a2_ref, a3_ref):
    f32 = jnp.float32

    acc = None
    for i in range(5):
        lhs = xt_ref[:, i:i + 32, :].reshape(B * 32, 108)
        d = jnp.dot(lhs, t1_ref[i], preferred_element_type=f32)
        acc = d if acc is None else acc + d
    y = jnp.maximum(acc * sc1_ref[...] + sh1_ref[...], 0.0)
    a2_ref[:, 0:2, :] = jnp.zeros((B, 2, 320), jnp.bfloat16)
    a2_ref[:, 18:20, :] = jnp.zeros((B, 2, 320), jnp.bfloat16)
    for g in range(NG):
        mv = _vpool(y[g * G * 32:(g + 1) * G * 32], sc1m_ref)
        p = _hpool_sel(mv, sel1_ref, rsel1_ref)
        a2_ref[g * G:(g + 1) * G, 2:18, :] = (
            p.astype(jnp.bfloat16).reshape(G, 16, 320))

    acc = None
    for i in range(5):
        lhs = a2_ref[:, i:i + 16, :].reshape(B * 16, 320)
        d = jnp.dot(lhs, t2_ref[i], preferred_element_type=f32)
        acc = d if acc is None else acc + d
    y = jnp.maximum(acc * sc2_ref[...] + sh2_ref[...], 0.0)
    a3_ref[:, 0:1, :] = jnp.zeros((B, 1, 320), f32)
    a3_ref[:, 9:10, :] = jnp.zeros((B, 1, 320), f32)
    for g in range(NG):
        mv = _vpool(y[g * G * 16:(g + 1) * G * 16], sc2m_ref)
        p = _hpool_sel(mv, sel2_ref, rsel2_ref)
        a3_ref[g * G:(g + 1) * G, 1:9, :] = p.reshape(G, 8, 320)

    acc = None
    for i in range(3):
        lhs = a3_ref[:, i:i + 8, :].reshape(B * 8, 320).astype(jnp.bfloat16)
        d = jnp.dot(lhs, t3_ref[i], preferred_element_type=f32)
        acc = d if acc is None else acc + d
    y = jnp.maximum(acc * sc3_ref[...] + sh3_ref[...], 0.0)
    hs = []
    for g in range(NG):
        mv = _vpool(y[g * G * 8:(g + 1) * G * 8], sc3m_ref)
        mr = jnp.dot(mv, r3_ref[...], preferred_element_type=f32)
        hs.append(jnp.maximum(mv.astype(f32), mr).astype(jnp.bfloat16))
    h = hs[0] if NG == 1 else jnp.concatenate(hs, axis=0)

    h = h.reshape(B, 4, 512)
    f = None
    for hh in range(4):
        d = jnp.dot(h[:, hh, :], wf1_ref[hh],
                    preferred_element_type=f32)
        f = d if f is None else f + d
    hrelu = jnp.maximum(f + fb1_ref[...], 0.0)
    o_ref[...] = jnp.dot(hrelu, fw2_ref[...],
                         preferred_element_type=f32) + fb2_ref[...]


def kernel(x, w1, s1, t1, w2, s2, t2, w3, s3, t3, fw1, fb1, fw2, fb2):
    n = x.shape[0]

    xt = jnp.transpose(x, (0, 2, 3, 1))
    xt = jnp.pad(xt, ((0, 0), (2, 2), (2, 2), (0, 0)))
    xt = xt.reshape(n, 36, 108).astype(jnp.bfloat16)

    T1 = _build_T(w1, *_L1)
    T2 = _build_T(w2, *_L2)
    T3 = _build_T(w3, *_L3)
    sc1, sh1 = jnp.tile(s1, (1, 32)), jnp.tile(t1, (1, 32))
    sc2, sh2 = jnp.tile(s2, (1, 16)), jnp.tile(t2, (1, 16))
    sc3, sh3 = jnp.tile(s3, (1, 8)), jnp.tile(t3, (1, 8))

    f4 = fw1.reshape(4, 4, 64, 256)
    Wf1 = jnp.zeros((4, 8, 64, 256), fw1.dtype).at[:, 0::2].set(f4)
    Wf1 = Wf1.reshape(4, 512, 256)

    bf = jnp.bfloat16
    sc1m, sc2m, sc3m = (jnp.asarray(_SC1, bf), jnp.asarray(_SC2, bf),
                        jnp.asarray(_SC3, bf))
    sel1, sel2 = jnp.asarray(_SEL1, bf), jnp.asarray(_SEL2, bf)
    rsel1, rsel2 = jnp.asarray(_RSEL1, bf), jnp.asarray(_RSEL2, bf)
    r3 = jnp.asarray(_R3, bf)

    full = lambda shape: pl.BlockSpec(shape, lambda i: tuple(0 for _ in shape))
    in_specs = [
        pl.BlockSpec((B, 36, 108), lambda i: (i, 0, 0)),
        full((5, 108, 512)), full((1, 512)), full((1, 512)),
        full((512, 512)), full((512, 320)), full((512, 320)),
        full((5, 320, 512)), full((1, 512)), full((1, 512)),
        full((256, 256)), full((512, 320)), full((512, 320)),
        full((3, 320, 512)), full((1, 512)), full((1, 512)),
        full((128, 128)), full((512, 512)),
        full((4, 512, 256)), full((1, 256)), full((256, 2)), full((1, 2)),
    ]
    out = pl.pallas_call(
        _cnn_body,
        out_shape=jax.ShapeDtypeStruct((n, 2), jnp.float32),
        grid=(n // B,),
        in_specs=in_specs,
        out_specs=pl.BlockSpec((B, 2), lambda i: (i, 0)),
        scratch_shapes=[
            pltpu.VMEM((B, 20, 320), jnp.bfloat16),
            pltpu.VMEM((B, 10, 320), jnp.float32),
        ],
        compiler_params=pltpu.CompilerParams(
            dimension_semantics=("arbitrary",)),
    )(xt, T1, sc1, sh1, sc1m, sel1, rsel1,
      T2, sc2, sh2, sc2m, sel2, rsel2,
      T3, sc3, sh3, sc3m, r3,
      Wf1, fb1, fw2, fb2)
    return out
